# baseline (speedup 1.0000x reference)
import functools

import jax
import jax.numpy as jnp
from jax import lax
from jax.experimental import pallas as pl
from jax.experimental.pallas import tpu as pltpu
from jax.experimental.pallas import tpu_sc as plsc

B = 16384
D = 64
NUM_REL = 1000
TABP = 1024
EPS = 1e-6
NC = 2
NS = 16
NW = NC * NS
RPW = B // NW
NCH = 4
CR = RPW // NCH
CGROUPS = CR // 16
DCH = 32


def _sc_body(s_hbm, o_hbm, rel_hbm, tab_hbm, out_hbm,
             idx_v, tab_sh, t_v, s0, o0, s1, o1, out_v,
             sem_t, sem0, sem1):
    cid = lax.axis_index("c")
    sid = lax.axis_index("s")
    wid = sid * NC + cid
    base = wid * RPW

    head = [pltpu.async_copy(rel_hbm.at[wid], idx_v, sem_t)]
    bufs = [(s0, o0, sem0), (s1, o1, sem1)]

    def start_pair(p, rb):
        sv, ov, sem = bufs[p]
        pltpu.async_copy(s_hbm.at[:, pl.ds(rb, CR)], sv, sem)
        pltpu.async_copy(o_hbm.at[:, pl.ds(rb, CR)], ov, sem)

    def drain_pair(p):
        sv, ov, sem = bufs[p]
        pltpu.make_async_copy(s_hbm.at[:, pl.ds(0, CR)], sv, sem).wait()
        pltpu.make_async_copy(o_hbm.at[:, pl.ds(0, CR)], ov, sem).wait()

    start_pair(0, base)
    start_pair(1, base + CR)

    with jax.named_scope("head_wait"):
        @pl.when(sid == 0)
        def _():
            pltpu.sync_copy(tab_hbm, tab_sh)

        plsc.subcore_barrier()
        pltpu.sync_copy(tab_sh, t_v)
        for h in head:
            h.wait()

    def round_body(r, carry):
        for p in (0, 1):
            c = r * 2 + p
            sv_ref, ov_ref, _ = bufs[p]
            with jax.named_scope(f"wait{p}"):
                drain_pair(p)

            scope = jax.named_scope(f"comp{p}")
            scope.__enter__()

            @plsc.parallel_loop(0, CGROUPS)
            def group(g, sv_ref=sv_ref, ov_ref=ov_ref, c=c):
                g16 = g * 16
                rel = idx_v[c, pl.ds(g16, 16)]

                def dstep(dd, accs):
                    accs = list(accs)
                    tb = rel + dd * (DCH * TABP)
                    for k in range(DCH):
                        d = dd * DCH + k
                        sd = sv_ref[d, pl.ds(g16, 16)]
                        od = ov_ref[d, pl.ds(g16, 16)]
                        td = plsc.load_gather(t_v, [tb + k * TABP])
                        df = sd + td - od + EPS
                        accs[k % 4] = accs[k % 4] + df * df
                    return tuple(accs)

                accs = lax.fori_loop(
                    0, D // DCH, dstep,
                    tuple(jnp.zeros((16,), jnp.float32) for _ in range(4)))
                acc = (accs[0] + accs[1]) + (accs[2] + accs[3])
                bits = lax.bitcast_convert_type(acc, jnp.int32)
                y = lax.bitcast_convert_type(
                    jnp.int32(0x5F3759DF) - (bits >> 1), jnp.float32)
                for _ in range(3):
                    y = y * (1.5 - 0.5 * acc * y * y)
                out_v[pl.ds(c * CR + g16, 16)] = acc * y

            scope.__exit__(None, None, None)

            @pl.when(r == 0)
            def _(p=p, c=c):
                start_pair(p, base + (c + 2) * CR)
        return carry

    lax.fori_loop(0, NCH // 2, round_body, 0)
    pltpu.sync_copy(out_v, out_hbm.at[pl.ds(base, RPW)])


_sc_call = functools.partial(
    pl.kernel,
    mesh=plsc.VectorSubcoreMesh(core_axis_name="c", subcore_axis_name="s"),
    out_type=jax.ShapeDtypeStruct((B,), jnp.float32),
    compiler_params=pltpu.CompilerParams(needs_layout_passes=False),
    scratch_types=[
        pltpu.VMEM((NCH, CR), jnp.int32),
        pltpu.VMEM_SHARED((D * TABP,), jnp.float32),
        pltpu.VMEM((D * TABP,), jnp.float32),
        pltpu.VMEM((D, CR), jnp.float32),
        pltpu.VMEM((D, CR), jnp.float32),
        pltpu.VMEM((D, CR), jnp.float32),
        pltpu.VMEM((D, CR), jnp.float32),
        pltpu.VMEM((RPW,), jnp.float32),
        pltpu.SemaphoreType.DMA,
        pltpu.SemaphoreType.DMA,
        pltpu.SemaphoreType.DMA,
    ],
)(_sc_body)


def kernel(subject_embeddings, object_embeddings, relations, relation_table):
    tab = jnp.pad(relation_table.T, ((0, 0), (0, TABP - NUM_REL)))
    return _sc_call(subject_embeddings.T, object_embeddings.T,
                    relations.astype(jnp.int32).reshape(NW, NCH, CR),
                    tab.reshape(D * TABP))


# speedup vs baseline: 1.0199x; 1.0199x over previous
import functools

import jax
import jax.numpy as jnp
from jax import lax
from jax.experimental import pallas as pl
from jax.experimental.pallas import tpu as pltpu
from jax.experimental.pallas import tpu_sc as plsc

B = 16384
D = 64
NUM_REL = 1000
TABP = 1024
EPS = 1e-6
NC = 2
NS = 16
NW = NC * NS
RPW = B // NW
NCH = 4
CR = RPW // NCH
CGROUPS = CR // 16
DCH = 16


def _sc_body(s_hbm, o_hbm, rel_hbm, tab_hbm, out_hbm,
             idx_v, tab_sh, t_v, s0, o0, s1, o1, out_v,
             sem_t, sem0, sem1):
    cid = lax.axis_index("c")
    sid = lax.axis_index("s")
    wid = sid * NC + cid
    base = wid * RPW

    head = [pltpu.async_copy(rel_hbm.at[wid], idx_v, sem_t)]
    bufs = [(s0, o0, sem0), (s1, o1, sem1)]

    def start_pair(p, rb):
        sv, ov, sem = bufs[p]
        pltpu.async_copy(s_hbm.at[:, pl.ds(rb, CR)], sv, sem)
        pltpu.async_copy(o_hbm.at[:, pl.ds(rb, CR)], ov, sem)

    def drain_pair(p):
        sv, ov, sem = bufs[p]
        pltpu.make_async_copy(s_hbm.at[:, pl.ds(0, CR)], sv, sem).wait()
        pltpu.make_async_copy(o_hbm.at[:, pl.ds(0, CR)], ov, sem).wait()

    start_pair(0, base)
    start_pair(1, base + CR)

    with jax.named_scope("head_wait"):
        @pl.when(sid == 0)
        def _():
            pltpu.sync_copy(tab_hbm, tab_sh)

        plsc.subcore_barrier()
        pltpu.sync_copy(tab_sh, t_v)
        for h in head:
            h.wait()

    def round_body(r, carry):
        for p in (0, 1):
            c = r * 2 + p
            sv_ref, ov_ref, _ = bufs[p]
            with jax.named_scope(f"wait{p}"):
                drain_pair(p)

            scope = jax.named_scope(f"comp{p}")
            scope.__enter__()

            @plsc.parallel_loop(0, CGROUPS)
            def group(g, sv_ref=sv_ref, ov_ref=ov_ref, c=c):
                g16 = g * 16
                rel = idx_v[c, pl.ds(g16, 16)]

                def dstep(dd, accs):
                    accs = list(accs)
                    tb = rel + dd * (DCH * TABP)
                    for k in range(DCH):
                        d = dd * DCH + k
                        sd = sv_ref[d, pl.ds(g16, 16)]
                        od = ov_ref[d, pl.ds(g16, 16)]
                        td = plsc.load_gather(t_v, [tb + k * TABP])
                        df = sd + td - od + EPS
                        accs[k % 4] = accs[k % 4] + df * df
                    return tuple(accs)

                accs = lax.fori_loop(
                    0, D // DCH, dstep,
                    tuple(jnp.zeros((16,), jnp.float32) for _ in range(4)))
                acc = (accs[0] + accs[1]) + (accs[2] + accs[3])
                bits = lax.bitcast_convert_type(acc, jnp.int32)
                y = lax.bitcast_convert_type(
                    jnp.int32(0x5F3759DF) - (bits >> 1), jnp.float32)
                for _ in range(3):
                    y = y * (1.5 - 0.5 * acc * y * y)
                out_v[pl.ds(c * CR + g16, 16)] = acc * y

            scope.__exit__(None, None, None)

            @pl.when(r == 0)
            def _(p=p, c=c):
                start_pair(p, base + (c + 2) * CR)
        return carry

    lax.fori_loop(0, NCH // 2, round_body, 0)
    pltpu.sync_copy(out_v, out_hbm.at[pl.ds(base, RPW)])


_sc_call = functools.partial(
    pl.kernel,
    mesh=plsc.VectorSubcoreMesh(core_axis_name="c", subcore_axis_name="s"),
    out_type=jax.ShapeDtypeStruct((B,), jnp.float32),
    compiler_params=pltpu.CompilerParams(needs_layout_passes=False),
    scratch_types=[
        pltpu.VMEM((NCH, CR), jnp.int32),
        pltpu.VMEM_SHARED((D * TABP,), jnp.float32),
        pltpu.VMEM((D * TABP,), jnp.float32),
        pltpu.VMEM((D, CR), jnp.float32),
        pltpu.VMEM((D, CR), jnp.float32),
        pltpu.VMEM((D, CR), jnp.float32),
        pltpu.VMEM((D, CR), jnp.float32),
        pltpu.VMEM((RPW,), jnp.float32),
        pltpu.SemaphoreType.DMA,
        pltpu.SemaphoreType.DMA,
        pltpu.SemaphoreType.DMA,
    ],
)(_sc_body)


def kernel(subject_embeddings, object_embeddings, relations, relation_table):
    tab = jnp.pad(relation_table.T, ((0, 0), (0, TABP - NUM_REL)))
    return _sc_call(subject_embeddings.T, object_embeddings.T,
                    relations.astype(jnp.int32).reshape(NW, NCH, CR),
                    tab.reshape(D * TABP))


# speedup vs baseline: 1.0246x; 1.0046x over previous
import functools

import jax
import jax.numpy as jnp
from jax import lax
from jax.experimental import pallas as pl
from jax.experimental.pallas import tpu as pltpu
from jax.experimental.pallas import tpu_sc as plsc

B = 16384
D = 64
NUM_REL = 1000
TABP = 1024
EPS = 1e-6
NC = 2
NS = 16
NW = NC * NS
RPW = B // NW
NCH = 4
CR = RPW // NCH
CGROUPS = CR // 16
DCH = 16


def _sc_body(s_hbm, o_hbm, rel_hbm, tab_hbm, out_hbm,
             idx_v, tab_sh, t_v, s0, o0, s1, o1, out_v,
             sem_t, sem0, sem1):
    cid = lax.axis_index("c")
    sid = lax.axis_index("s")
    wid = sid * NC + cid
    base = wid * RPW

    head = [pltpu.async_copy(rel_hbm.at[wid], idx_v, sem_t)]
    bufs = [(s0, o0, sem0), (s1, o1, sem1)]

    def start_pair(p, rb):
        sv, ov, sem = bufs[p]
        pltpu.async_copy(s_hbm.at[:, pl.ds(rb, CR)], sv, sem)
        pltpu.async_copy(o_hbm.at[:, pl.ds(rb, CR)], ov, sem)

    def drain_pair(p):
        sv, ov, sem = bufs[p]
        pltpu.make_async_copy(s_hbm.at[:, pl.ds(0, CR)], sv, sem).wait()
        pltpu.make_async_copy(o_hbm.at[:, pl.ds(0, CR)], ov, sem).wait()

    start_pair(0, base)
    start_pair(1, base + CR)

    @pl.when(sid == 0)
    def _():
        pltpu.sync_copy(tab_hbm, tab_sh)

    plsc.subcore_barrier()
    pltpu.sync_copy(tab_sh, t_v)
    for h in head:
        h.wait()

    def round_body(r, carry):
        for p in (0, 1):
            c = r * 2 + p
            sv_ref, ov_ref, _ = bufs[p]
            drain_pair(p)

            @plsc.parallel_loop(0, CGROUPS)
            def group(g, sv_ref=sv_ref, ov_ref=ov_ref, c=c):
                g16 = g * 16
                rel = idx_v[c, pl.ds(g16, 16)]

                def dstep(dd, accs):
                    accs = list(accs)
                    tb = rel + dd * (DCH * TABP)
                    for k in range(DCH):
                        d = dd * DCH + k
                        sd = sv_ref[d, pl.ds(g16, 16)]
                        od = ov_ref[d, pl.ds(g16, 16)]
                        td = plsc.load_gather(t_v, [tb + k * TABP])
                        df = sd + td - od + EPS
                        accs[k % 4] = accs[k % 4] + df * df
                    return tuple(accs)

                accs = lax.fori_loop(
                    0, D // DCH, dstep,
                    tuple(jnp.zeros((16,), jnp.float32) for _ in range(4)))
                acc = (accs[0] + accs[1]) + (accs[2] + accs[3])
                bits = lax.bitcast_convert_type(acc, jnp.int32)
                y = lax.bitcast_convert_type(
                    jnp.int32(0x5F3759DF) - (bits >> 1), jnp.float32)
                for _ in range(3):
                    y = y * (1.5 - 0.5 * acc * y * y)
                out_v[pl.ds(c * CR + g16, 16)] = acc * y

            @pl.when(r == 0)
            def _(p=p, c=c):
                start_pair(p, base + (c + 2) * CR)
        return carry

    lax.fori_loop(0, NCH // 2, round_body, 0)
    pltpu.sync_copy(out_v, out_hbm.at[pl.ds(base, RPW)])


_sc_call = functools.partial(
    pl.kernel,
    mesh=plsc.VectorSubcoreMesh(core_axis_name="c", subcore_axis_name="s"),
    out_type=jax.ShapeDtypeStruct((B,), jnp.float32),
    compiler_params=pltpu.CompilerParams(needs_layout_passes=False),
    scratch_types=[
        pltpu.VMEM((NCH, CR), jnp.int32),
        pltpu.VMEM_SHARED((D * TABP,), jnp.float32),
        pltpu.VMEM((D * TABP,), jnp.float32),
        pltpu.VMEM((D, CR), jnp.float32),
        pltpu.VMEM((D, CR), jnp.float32),
        pltpu.VMEM((D, CR), jnp.float32),
        pltpu.VMEM((D, CR), jnp.float32),
        pltpu.VMEM((RPW,), jnp.float32),
        pltpu.SemaphoreType.DMA,
        pltpu.SemaphoreType.DMA,
        pltpu.SemaphoreType.DMA,
    ],
)(_sc_body)


def kernel(subject_embeddings, object_embeddings, relations, relation_table):
    tab = jnp.pad(relation_table.T, ((0, 0), (0, TABP - NUM_REL)))
    return _sc_call(subject_embeddings.T, object_embeddings.T,
                    relations.astype(jnp.int32).reshape(NW, NCH, CR),
                    tab.reshape(D * TABP))
